# baseline (speedup 1.0000x reference)
import numpy as np
import jax
import jax.numpy as jnp
from jax.experimental import pallas as pl
from jax.experimental.pallas import tpu as pltpu

_H = 512
_W = 512
_NPIX = float(_H * _W)
_NUM_BINS = 8
_WIDTH = 255.0 / 8.0

_FR = float(-np.sin(2.0 * np.pi * 5 / 8))
_FC = float(np.cos(2.0 * np.pi * 5 / 8) + 1.0)
_A = _FR * _FC
_B = _FR * _FR
_T = 1.0 - (1.0 - _FR) * _FC
_AT = np.float32(_A / _T)
_BT = np.float32(_B / _T)

_MOB = np.zeros((8, _NUM_BINS), dtype=np.float32)
_MOB[0] = [-1.0, 1.0, 0.0, 0.0, 0.0, 0.0, 0.0, 0.0]
_MOB[1] = [-1.0, 0.0, 1.0, 0.0, 0.0, 0.0, 0.0, 0.0]
_MOB[2] = [-1.0, 0.0, 0.0, 0.0, 1.0, 0.0, 0.0, 0.0]
_MOB[3] = [1.0, -1.0, -1.0, 1.0, 0.0, 0.0, 0.0, 0.0]
_MOB[4] = [1.0, -1.0, 0.0, 0.0, -1.0, 1.0, 0.0, 0.0]
_MOB[5] = [1.0, 0.0, -1.0, 0.0, -1.0, 0.0, 1.0, 0.0]
_MOB[6] = [-1.0, 1.0, 1.0, -1.0, 1.0, -1.0, -1.0, 1.0]
_MOB[7] = [1.0, 0.0, 0.0, 0.0, 0.0, 0.0, 0.0, 0.0]


def _lbp_hist_kernel(x_ref, mob_ref, out_ref):
    x = x_ref[0]
    zrow = jnp.zeros((1, _W), jnp.float32)
    zcol = jnp.zeros((_H, 1), jnp.float32)
    rn = jnp.concatenate([x[1:, :], zrow], axis=0)
    g = _AT * x + _BT * rn
    gm = jnp.concatenate([zcol, g[:, :-1]], axis=1)
    gp = jnp.concatenate([g[:, 1:], zcol], axis=1)
    w = x - _AT * rn

    m5 = gm >= w
    m6 = rn >= x
    m7 = gp >= w
    one = jnp.float32(1.0)
    zero = jnp.float32(0.0)
    b5 = jnp.where(m5, one, zero)
    b6 = jnp.where(m6, one, zero)
    b7 = jnp.where(m7, one, zero)
    p56 = jnp.where(m5, b6, zero)
    p57 = jnp.where(m5, b7, zero)
    p67 = jnp.where(m6, b7, zero)
    p567 = jnp.where(m5, p67, zero)

    mob = mob_ref[...]
    counts = (
        jnp.sum(b5) * mob[0]
        + jnp.sum(b6) * mob[1]
        + jnp.sum(b7) * mob[2]
        + jnp.sum(p56) * mob[3]
        + jnp.sum(p57) * mob[4]
        + jnp.sum(p67) * mob[5]
        + jnp.sum(p567) * mob[6]
        + _NPIX * mob[7]
    )
    dens = (counts / np.float32(_WIDTH)) / np.float32(_NPIX)
    out_ref[0, 0] = dens


def kernel(x):
    B, C, H, W = x.shape
    planes = x.reshape(B * C, H, W)
    out = pl.pallas_call(
        _lbp_hist_kernel,
        grid=(B * C,),
        in_specs=[
            pl.BlockSpec((1, H, W), lambda i: (i, 0, 0)),
            pl.BlockSpec((8, _NUM_BINS), lambda i: (0, 0)),
        ],
        out_specs=pl.BlockSpec((1, 1, _NUM_BINS), lambda i: (i, 0, 0)),
        out_shape=jax.ShapeDtypeStruct((B * C, 1, _NUM_BINS), jnp.float32),
        compiler_params=pltpu.CompilerParams(
            dimension_semantics=("parallel",),
        ),
    )(planes, jnp.asarray(_MOB))
    return out.reshape(B, C * _NUM_BINS)


# speedup vs baseline: 1.0308x; 1.0308x over previous
import numpy as np
import jax
import jax.numpy as jnp
from jax.experimental import pallas as pl
from jax.experimental.pallas import tpu as pltpu

_H = 512
_W = 512
_NPIX = float(_H * _W)
_NUM_BINS = 8
_WIDTH = 255.0 / 8.0

_FR = float(-np.sin(2.0 * np.pi * 5 / 8))
_FC = float(np.cos(2.0 * np.pi * 5 / 8) + 1.0)
_A = _FR * _FC
_B = _FR * _FR
_T = 1.0 - (1.0 - _FR) * _FC
_AT = np.float32(_A / _T)
_BT = np.float32(_B / _T)

_MOB = np.zeros((8, _NUM_BINS), dtype=np.float32)
_MOB[0] = [-1.0, 1.0, 0.0, 0.0, 0.0, 0.0, 0.0, 0.0]
_MOB[1] = [-1.0, 0.0, 1.0, 0.0, 0.0, 0.0, 0.0, 0.0]
_MOB[2] = [-1.0, 0.0, 0.0, 0.0, 1.0, 0.0, 0.0, 0.0]
_MOB[3] = [1.0, -1.0, -1.0, 1.0, 0.0, 0.0, 0.0, 0.0]
_MOB[4] = [1.0, -1.0, 0.0, 0.0, -1.0, 1.0, 0.0, 0.0]
_MOB[5] = [1.0, 0.0, -1.0, 0.0, -1.0, 0.0, 1.0, 0.0]
_MOB[6] = [-1.0, 1.0, 1.0, -1.0, 1.0, -1.0, -1.0, 1.0]
_MOB[7] = [1.0, 0.0, 0.0, 0.0, 0.0, 0.0, 0.0, 0.0]


def _lbp_hist_kernel(x_ref, mob_ref, out_ref):
    x = x_ref[0]
    zrow = jnp.zeros((1, _W), jnp.float32)
    zcol = jnp.zeros((_H, 1), jnp.float32)
    rn = jnp.concatenate([x[1:, :], zrow], axis=0)
    g = _AT * x + _BT * rn
    gm = jnp.concatenate([zcol, g[:, :-1]], axis=1)
    gp = jnp.concatenate([g[:, 1:], zcol], axis=1)
    w = x - _AT * rn

    b5 = (gm >= w).astype(jnp.float32)
    b6 = (rn >= x).astype(jnp.float32)
    b7 = (gp >= w).astype(jnp.float32)
    p56 = b5 * b6
    p57 = b5 * b7
    p67 = b6 * b7
    p567 = p56 * b7

    mob = mob_ref[...]
    counts = (
        jnp.sum(b5) * mob[0]
        + jnp.sum(b6) * mob[1]
        + jnp.sum(b7) * mob[2]
        + jnp.sum(p56) * mob[3]
        + jnp.sum(p57) * mob[4]
        + jnp.sum(p67) * mob[5]
        + jnp.sum(p567) * mob[6]
        + _NPIX * mob[7]
    )
    dens = (counts / np.float32(_WIDTH)) / np.float32(_NPIX)
    out_ref[0, 0] = dens


def kernel(x):
    B, C, H, W = x.shape
    planes = x.reshape(B * C, H, W)
    out = pl.pallas_call(
        _lbp_hist_kernel,
        grid=(B * C,),
        in_specs=[
            pl.BlockSpec((1, H, W), lambda i: (i, 0, 0)),
            pl.BlockSpec((8, _NUM_BINS), lambda i: (0, 0)),
        ],
        out_specs=pl.BlockSpec((1, 1, _NUM_BINS), lambda i: (i, 0, 0)),
        out_shape=jax.ShapeDtypeStruct((B * C, 1, _NUM_BINS), jnp.float32),
        compiler_params=pltpu.CompilerParams(
            dimension_semantics=("parallel",),
        ),
    )(planes, jnp.asarray(_MOB))
    return out.reshape(B, C * _NUM_BINS)


# speedup vs baseline: 1.0548x; 1.0232x over previous
import numpy as np
import jax
import jax.numpy as jnp
from jax.experimental import pallas as pl
from jax.experimental.pallas import tpu as pltpu

_H = 512
_W = 512
_NPIX = float(_H * _W)
_NUM_BINS = 8
_WIDTH = 255.0 / 8.0

_FR = float(-np.sin(2.0 * np.pi * 5 / 8))
_FC = float(np.cos(2.0 * np.pi * 5 / 8) + 1.0)
_A = _FR * _FC
_B = _FR * _FR
_T = 1.0 - (1.0 - _FR) * _FC
_AT = np.float32(_A / _T)
_BT = np.float32(_B / _T)

_MOB = np.zeros((8, _NUM_BINS), dtype=np.float32)
_MOB[0] = [-1.0, 1.0, 0.0, 0.0, 0.0, 0.0, 0.0, 0.0]
_MOB[1] = [-1.0, 0.0, 1.0, 0.0, 0.0, 0.0, 0.0, 0.0]
_MOB[2] = [-1.0, 0.0, 0.0, 0.0, 1.0, 0.0, 0.0, 0.0]
_MOB[3] = [1.0, -1.0, -1.0, 1.0, 0.0, 0.0, 0.0, 0.0]
_MOB[4] = [1.0, -1.0, 0.0, 0.0, -1.0, 1.0, 0.0, 0.0]
_MOB[5] = [1.0, 0.0, -1.0, 0.0, -1.0, 0.0, 1.0, 0.0]
_MOB[6] = [-1.0, 1.0, 1.0, -1.0, 1.0, -1.0, -1.0, 1.0]
_MOB[7] = [1.0, 0.0, 0.0, 0.0, 0.0, 0.0, 0.0, 0.0]


_PLANES_PER_STEP = 2


def _plane_hist(x, masks, mob):
    last_row, first_col, last_col = masks
    rn = pltpu.roll(x, _H - 1, 0) * last_row
    g = _AT * x + _BT * rn
    gm = pltpu.roll(g, 1, 1) * first_col
    gp = pltpu.roll(g, _W - 1, 1) * last_col
    w = x - _AT * rn

    b5 = (gm >= w).astype(jnp.float32)
    b6 = (rn >= x).astype(jnp.float32)
    b7 = (gp >= w).astype(jnp.float32)
    p56 = b5 * b6
    p57 = b5 * b7
    p67 = b6 * b7
    p567 = p56 * b7

    counts = (
        jnp.sum(b5) * mob[0]
        + jnp.sum(b6) * mob[1]
        + jnp.sum(b7) * mob[2]
        + jnp.sum(p56) * mob[3]
        + jnp.sum(p57) * mob[4]
        + jnp.sum(p67) * mob[5]
        + jnp.sum(p567) * mob[6]
        + _NPIX * mob[7]
    )
    return (counts / np.float32(_WIDTH)) / np.float32(_NPIX)


def _lbp_hist_kernel(x_ref, mob_ref, out_ref):
    rowi = jax.lax.broadcasted_iota(jnp.int32, (_H, 1), 0)
    coli = jax.lax.broadcasted_iota(jnp.int32, (1, _W), 1)
    masks = (
        jnp.where(rowi < _H - 1, 1.0, 0.0).astype(jnp.float32),
        jnp.where(coli > 0, 1.0, 0.0).astype(jnp.float32),
        jnp.where(coli < _W - 1, 1.0, 0.0).astype(jnp.float32),
    )
    mob = mob_ref[...]
    for k in range(_PLANES_PER_STEP):
        out_ref[k, 0] = _plane_hist(x_ref[k], masks, mob)


def kernel(x):
    B, C, H, W = x.shape
    planes = x.reshape(B * C, H, W)
    n_steps = (B * C) // _PLANES_PER_STEP
    out = pl.pallas_call(
        _lbp_hist_kernel,
        grid=(n_steps,),
        in_specs=[
            pl.BlockSpec((_PLANES_PER_STEP, H, W), lambda i: (i, 0, 0)),
            pl.BlockSpec((8, _NUM_BINS), lambda i: (0, 0)),
        ],
        out_specs=pl.BlockSpec(
            (_PLANES_PER_STEP, 1, _NUM_BINS), lambda i: (i, 0, 0)),
        out_shape=jax.ShapeDtypeStruct((B * C, 1, _NUM_BINS), jnp.float32),
        compiler_params=pltpu.CompilerParams(
            dimension_semantics=("parallel",),
        ),
    )(planes, jnp.asarray(_MOB))
    return out.reshape(B, C * _NUM_BINS)
